# baseline (speedup 1.0000x reference)
import functools

import jax
import jax.numpy as jnp
from jax import lax
from jax.experimental import pallas as pl
from jax.experimental.pallas import tpu as pltpu
from jax.experimental.pallas import tpu_sc as plsc

VOCAB = 100000
D_MODEL = 1024
MAX_LEN = 4096
BATCH = 128
Q_LEN = 32

_L = 16
_NVEC = D_MODEL // _L
_B = BATCH * Q_LEN
_GCH = 32
_NGCH = 4
_NIN = 2
_WCH = 16
_NOUT = 3
_CBLK = 8


def _sc_kernel_call(table, y3, alpha16, scale16, pe2, ipl1):
    info = plsc.get_sparse_core_info()
    nc, ns = info.num_cores, info.num_subcores
    nw = nc * ns
    rows_per_w = _B // nw
    assert rows_per_w == _NGCH * _GCH

    mesh = plsc.VectorSubcoreMesh(core_axis_name="c", subcore_axis_name="s")

    @functools.partial(
        pl.kernel,
        mesh=mesh,
        out_type=jax.ShapeDtypeStruct((_B, D_MODEL), jnp.float32),
        scratch_types=(
            [pltpu.VMEM((_NGCH, _GCH), jnp.int32)]
            + [pltpu.VMEM((_GCH, D_MODEL), jnp.float32) for _ in range(_NIN)]
            + [pltpu.VMEM((_WCH, D_MODEL), jnp.float32) for _ in range(_NOUT)]
            + [pltpu.VMEM((1,), jnp.int32),
               pltpu.VMEM((1, D_MODEL), jnp.float32),
               pltpu.VMEM((D_MODEL,), jnp.float32),
               pltpu.VMEM((_L,), jnp.float32),
               pltpu.VMEM((_L,), jnp.float32)]
            + [pltpu.SemaphoreType.DMA for _ in range(_NIN + _NOUT + 1)]
        ),
    )
    def k(table_hbm, y_hbm, alpha_hbm, scale_hbm, pe_hbm, ipl_hbm, out_hbm,
          *refs):
        idx_v = refs[0]
        ibufs = refs[1:1 + _NIN]
        obufs = refs[1 + _NIN:1 + _NIN + _NOUT]
        nb = 1 + _NIN + _NOUT
        idx1, pos1, spos, alpha_v, scale_v = refs[nb:nb + 5]
        gsems = refs[nb + 5:nb + 5 + _NIN]
        wsems = refs[nb + 5 + _NIN:nb + 5 + _NIN + _NOUT]
        psem = refs[nb + 5 + _NIN + _NOUT]

        wid = lax.axis_index("s") * nc + lax.axis_index("c")
        base = wid * rows_per_w

        pltpu.sync_copy(y_hbm.at[wid], idx_v)

        def start_gather(g, s):
            return pltpu.async_copy(table_hbm.at[idx_v.at[g]], ibufs[s],
                                    gsems[s])

        def start_write(h, s):
            return pltpu.async_copy(obufs[s],
                                    out_hbm.at[pl.ds(base + h * _WCH, _WCH)],
                                    wsems[s])

        inflight_g = [start_gather(g, g) for g in range(_NIN)]

        pltpu.sync_copy(ipl_hbm, idx1)
        pltpu.async_copy(pe_hbm.at[idx1], pos1, psem).wait()
        pltpu.async_copy(alpha_hbm, alpha_v, psem).wait()
        pltpu.async_copy(scale_hbm, scale_v, psem).wait()

        av = alpha_v[...]
        sv = scale_v[...]

        def scale_pos(j, carry):
            spos[pl.ds(j * _L, _L)] = pos1[0, pl.ds(j * _L, _L)] * av
            return carry

        lax.fori_loop(0, _NVEC, scale_pos, 0)

        def compute(si, half, so):
            src = ibufs[si]
            dst = obufs[so]
            r0 = half * _WCH
            for b in range(_NVEC // _CBLK):
                pvs = [spos[pl.ds((b * _CBLK + j) * _L, _L)]
                       for j in range(_CBLK)]

                def rows_body(r, carry, b=b, pvs=pvs, r0=r0):
                    for j in range(_CBLK):
                        sl = pl.ds((b * _CBLK + j) * _L, _L)
                        dst[r, sl] = src[r0 + r, sl] * sv + pvs[j]
                    return carry

                lax.fori_loop(0, _WCH, rows_body, 0)

        inflight_w = [None] * _NOUT
        nhalf = _NGCH * 2
        for h in range(nhalf):
            g = h // 2
            si = g % _NIN
            half = h % 2
            so = h % _NOUT
            if half == 0:
                inflight_g[si].wait()
            if inflight_w[so] is not None:
                inflight_w[so].wait()
                inflight_w[so] = None
            compute(si, half, so)
            inflight_w[so] = start_write(h, so)
            if half == 1 and g + _NIN < _NGCH:
                inflight_g[si] = start_gather(g + _NIN, si)
        for s in range(_NOUT):
            if inflight_w[s] is not None:
                inflight_w[s].wait()

    return k(table, y3, alpha16, scale16, pe2, ipl1)


def kernel(emb_table, alpha, pe, x_scale, y, idx_plus_len):
    y3 = y.reshape(32, _NGCH, _GCH).astype(jnp.int32)
    pe2 = pe.reshape(MAX_LEN, D_MODEL)
    ipl1 = jnp.asarray(idx_plus_len, jnp.int32).reshape(1)
    alpha16 = jnp.broadcast_to(alpha.astype(jnp.float32), (_L,))
    scale16 = jnp.broadcast_to(jnp.asarray(x_scale, jnp.float32), (_L,))

    out = _sc_kernel_call(emb_table, y3, alpha16, scale16, pe2, ipl1)
    return out.reshape(BATCH, Q_LEN, D_MODEL)


# speedup vs baseline: 1.0397x; 1.0397x over previous
import functools

import jax
import jax.numpy as jnp
from jax import lax
from jax.experimental import pallas as pl
from jax.experimental.pallas import tpu as pltpu
from jax.experimental.pallas import tpu_sc as plsc

VOCAB = 100000
D_MODEL = 1024
BATCH = 128
Q_LEN = 32

_L = 16
_NVEC = D_MODEL // _L
_B = BATCH * Q_LEN
_NIN = 4
_NOUT = 3
_CH = 16
_NCH = 8
_CBLK = 8


def _sc_kernel_call(table, y3, alpha16, scale16, pos):
    info = plsc.get_sparse_core_info()
    nc, ns = info.num_cores, info.num_subcores
    nw = nc * ns
    rows_per_w = _B // nw
    assert rows_per_w == _NCH * _CH

    mesh = plsc.VectorSubcoreMesh(core_axis_name="c", subcore_axis_name="s")

    @functools.partial(
        pl.kernel,
        mesh=mesh,
        out_type=jax.ShapeDtypeStruct((_B, D_MODEL), jnp.float32),
        scratch_types=(
            [pltpu.VMEM((_NCH, _CH), jnp.int32)]
            + [pltpu.VMEM((_CH, D_MODEL), jnp.float32)
               for _ in range(_NIN + _NOUT)]
            + [pltpu.VMEM((D_MODEL,), jnp.float32),
               pltpu.VMEM((D_MODEL,), jnp.float32),
               pltpu.VMEM((_L,), jnp.float32),
               pltpu.VMEM((_L,), jnp.float32)]
            + [pltpu.SemaphoreType.DMA for _ in range(_NIN + _NOUT + 1)]
        ),
    )
    def k(table_hbm, y_hbm, alpha_hbm, scale_hbm, pos_hbm, out_hbm, *refs):
        idx_v = refs[0]
        ibufs = refs[1:1 + _NIN]
        obufs = refs[1 + _NIN:1 + _NIN + _NOUT]
        pos_v, spos, alpha_v, scale_v = refs[1 + _NIN + _NOUT:5 + _NIN + _NOUT]
        gsems = refs[5 + _NIN + _NOUT:5 + 2 * _NIN + _NOUT]
        wsems = refs[5 + 2 * _NIN + _NOUT:5 + 2 * _NIN + 2 * _NOUT]
        psem = refs[5 + 2 * _NIN + 2 * _NOUT]

        wid = lax.axis_index("s") * nc + lax.axis_index("c")
        base = wid * rows_per_w

        pltpu.sync_copy(y_hbm.at[wid], idx_v)

        def start_gather(c, s):
            return pltpu.async_copy(table_hbm.at[idx_v.at[c]], ibufs[s],
                                    gsems[s])

        def start_write(c, s):
            return pltpu.async_copy(obufs[s],
                                    out_hbm.at[pl.ds(base + c * _CH, _CH)],
                                    wsems[s])

        inflight_g = [None] * _NIN
        for c in range(_NIN - 1):
            inflight_g[c] = start_gather(c, c)

        pltpu.async_copy(alpha_hbm, alpha_v, psem).wait()
        pltpu.async_copy(scale_hbm, scale_v, psem).wait()
        pltpu.async_copy(pos_hbm, pos_v, psem).wait()

        av = alpha_v[...]
        sv = scale_v[...]

        def scale_pos(j, carry):
            spos[pl.ds(j * _L, _L)] = pos_v[pl.ds(j * _L, _L)] * av
            return carry

        lax.fori_loop(0, _NVEC, scale_pos, 0)

        def compute(si, so):
            src = ibufs[si]
            dst = obufs[so]
            for b in range(_NVEC // _CBLK):
                pvs = [spos[pl.ds((b * _CBLK + j) * _L, _L)]
                       for j in range(_CBLK)]

                def rows_body(r2, carry, b=b, pvs=pvs):
                    r = r2 * 2
                    for j in range(_CBLK):
                        sl = pl.ds((b * _CBLK + j) * _L, _L)
                        dst[r, sl] = src[r, sl] * sv + pvs[j]
                        dst[r + 1, sl] = src[r + 1, sl] * sv + pvs[j]
                    return carry

                lax.fori_loop(0, _CH // 2, rows_body, 0)

        inflight_w = [None] * _NOUT
        for i in range(_NCH):
            si = i % _NIN
            so = i % _NOUT
            j = i + _NIN - 1
            if j < _NCH:
                inflight_g[j % _NIN] = start_gather(j, j % _NIN)
            inflight_g[si].wait()
            if inflight_w[so] is not None:
                inflight_w[so].wait()
                inflight_w[so] = None
            compute(si, so)
            inflight_w[so] = start_write(i, so)
        for s in range(_NOUT):
            if inflight_w[s] is not None:
                inflight_w[s].wait()

    return k(table, y3, alpha16, scale16, pos)


def kernel(emb_table, alpha, pe, x_scale, y, idx_plus_len):
    y_flat = y.reshape(-1).astype(jnp.int32)
    y3 = y_flat.reshape(32, _NCH, _CH)
    pos = lax.dynamic_index_in_dim(pe[0], idx_plus_len, axis=0,
                                   keepdims=False)
    alpha16 = jnp.broadcast_to(alpha.astype(jnp.float32), (_L,))
    scale16 = jnp.broadcast_to(jnp.asarray(x_scale, jnp.float32), (_L,))

    out = _sc_kernel_call(emb_table, y3, alpha16, scale16, pos)
    return out.reshape(BATCH, Q_LEN, D_MODEL)


# speedup vs baseline: 1.4713x; 1.4151x over previous
import functools

import jax
import jax.numpy as jnp
from jax import lax
from jax.experimental import pallas as pl
from jax.experimental.pallas import tpu as pltpu
from jax.experimental.pallas import tpu_sc as plsc

VOCAB = 100000
D_MODEL = 1024
BATCH = 128
Q_LEN = 32

_L = 16
_NVEC = D_MODEL // _L
_B = BATCH * Q_LEN
_NIN = 4
_NOUT = 3
_CH = 16
_NCH = 8
_CBLK = 8


def _sc_kernel_call(table, y3, alpha16, scale16, pos):
    info = plsc.get_sparse_core_info()
    nc, ns = info.num_cores, info.num_subcores
    nw = nc * ns
    rows_per_w = _B // nw
    assert rows_per_w == _NCH * _CH

    mesh = plsc.VectorSubcoreMesh(core_axis_name="c", subcore_axis_name="s")

    @functools.partial(
        pl.kernel,
        mesh=mesh,
        out_type=jax.ShapeDtypeStruct((_B, D_MODEL), jnp.float32),
        scratch_types=(
            [pltpu.VMEM((_NCH, _CH), jnp.int32)]
            + [pltpu.VMEM((_CH, D_MODEL), jnp.float32)
               for _ in range(_NIN + _NOUT)]
            + [pltpu.VMEM((D_MODEL,), jnp.float32),
               pltpu.VMEM((D_MODEL,), jnp.float32),
               pltpu.VMEM((_L,), jnp.float32),
               pltpu.VMEM((_L,), jnp.float32)]
            + [pltpu.SemaphoreType.DMA for _ in range(_NIN + _NOUT + 1)]
        ),
    )
    def k(table_hbm, y_hbm, alpha_hbm, scale_hbm, pos_hbm, out_hbm, *refs):
        idx_v = refs[0]
        ibufs = refs[1:1 + _NIN]
        obufs = refs[1 + _NIN:1 + _NIN + _NOUT]
        pos_v, spos, alpha_v, scale_v = refs[1 + _NIN + _NOUT:5 + _NIN + _NOUT]
        gsems = refs[5 + _NIN + _NOUT:5 + 2 * _NIN + _NOUT]
        wsems = refs[5 + 2 * _NIN + _NOUT:5 + 2 * _NIN + 2 * _NOUT]
        psem = refs[5 + 2 * _NIN + 2 * _NOUT]

        wid = lax.axis_index("s") * nc + lax.axis_index("c")
        base = wid * rows_per_w

        pltpu.sync_copy(y_hbm.at[wid], idx_v)

        def start_gather(c, s):
            return pltpu.async_copy(table_hbm.at[idx_v.at[c]], ibufs[s],
                                    gsems[s])

        def start_write(c, s):
            return pltpu.async_copy(obufs[s],
                                    out_hbm.at[pl.ds(base + c * _CH, _CH)],
                                    wsems[s])

        inflight_g = [None] * _NIN
        for c in range(_NIN - 1):
            inflight_g[c] = start_gather(c, c)

        pltpu.async_copy(alpha_hbm, alpha_v, psem).wait()
        pltpu.async_copy(scale_hbm, scale_v, psem).wait()
        pltpu.async_copy(pos_hbm, pos_v, psem).wait()

        av = alpha_v[...]
        sv = scale_v[...]

        def scale_pos(j, carry):
            spos[pl.ds(j * _L, _L)] = pos_v[pl.ds(j * _L, _L)] * av
            return carry

        lax.fori_loop(0, _NVEC, scale_pos, 0)

        def compute(si, so):
            src = ibufs[si]
            dst = obufs[so]

            def block_body(b, carry):
                pvs = [spos[pl.ds((b * _CBLK + j) * _L, _L)]
                       for j in range(_CBLK)]

                def rows_body(r, carry2):
                    for j in range(_CBLK):
                        sl = pl.ds((b * _CBLK + j) * _L, _L)
                        dst[r, sl] = src[r, sl] * sv + pvs[j]
                    return carry2

                lax.fori_loop(0, _CH, rows_body, 0)
                return carry

            lax.fori_loop(0, _NVEC // _CBLK, block_body, 0)

        inflight_w = [None] * _NOUT
        for i in range(_NCH):
            si = i % _NIN
            so = i % _NOUT
            j = i + _NIN - 1
            if j < _NCH:
                inflight_g[j % _NIN] = start_gather(j, j % _NIN)
            inflight_g[si].wait()
            if inflight_w[so] is not None:
                inflight_w[so].wait()
                inflight_w[so] = None
            compute(si, so)
            inflight_w[so] = start_write(i, so)
        for s in range(_NOUT):
            if inflight_w[s] is not None:
                inflight_w[s].wait()

    return k(table, y3, alpha16, scale16, pos)


def kernel(emb_table, alpha, pe, x_scale, y, idx_plus_len):
    y_flat = y.reshape(-1).astype(jnp.int32)
    y3 = y_flat.reshape(32, _NCH, _CH)
    pos = lax.dynamic_index_in_dim(pe[0], idx_plus_len, axis=0,
                                   keepdims=False)
    alpha16 = jnp.broadcast_to(alpha.astype(jnp.float32), (_L,))
    scale16 = jnp.broadcast_to(jnp.asarray(x_scale, jnp.float32), (_L,))

    out = _sc_kernel_call(emb_table, y3, alpha16, scale16, pos)
    return out.reshape(BATCH, Q_LEN, D_MODEL)
